# baseline (speedup 1.0000x reference)
import functools

import jax
import jax.numpy as jnp
from jax import lax
from jax.experimental import pallas as pl
from jax.experimental.pallas import tpu as pltpu
from jax.experimental.pallas import tpu_sc as plsc

N = 10000
E = 320000
IN = 128
HD = 128
G = 64
F = 3

NC = 2
NS = 16
NW = NC * NS
EPT = E // NW
C = 64
NCHUNK = 160
EPTP = NCHUNK * C
NPAD = 10240
RPT = NPAD // NS

_mesh = plsc.VectorSubcoreMesh(core_axis_name="c", subcore_axis_name="s")


@functools.partial(
    pl.kernel,
    out_type=jax.ShapeDtypeStruct((NW * N,), jnp.float32),
    mesh=_mesh,
    scratch_types=[
        pltpu.VMEM((N,), jnp.float32),
        pltpu.VMEM((EPT,), jnp.int32),
    ],
    compiler_params=pltpu.CompilerParams(needs_layout_passes=False),
)
def _deg_kernel(dst_hbm, out_hbm, deg_v, idx_v):
    cid = lax.axis_index("c")
    sid = lax.axis_index("s")
    wid = cid * NS + sid

    zero16 = jnp.zeros((16,), jnp.float32)

    def zbody(i, carry):
        deg_v[pl.ds(i * 16, 16)] = zero16
        return carry

    lax.fori_loop(0, N // 16, zbody, 0)

    pltpu.sync_copy(dst_hbm.at[pl.ds(wid * EPT, EPT)], idx_v)

    ones16 = jnp.ones((16,), jnp.float32)

    def body(i, carry):
        idx16 = idx_v[pl.ds(i * 16, 16)]
        plsc.addupdate_scatter(deg_v, [idx16], ones16)
        return carry

    lax.fori_loop(0, EPT // 16, body, 0)
    pltpu.sync_copy(deg_v, out_hbm.at[pl.ds(wid * N, N)])


@functools.partial(
    pl.kernel,
    out_type=jax.ShapeDtypeStruct((NC, NPAD, HD), jnp.float32),
    mesh=_mesh,
    scratch_types=[
        pltpu.VMEM_SHARED((NPAD, HD), jnp.float32),
        pltpu.VMEM((C,), jnp.int32),
        pltpu.VMEM((C,), jnp.int32),
        pltpu.VMEM((C,), jnp.int32),
        pltpu.VMEM((C,), jnp.int32),
        pltpu.VMEM((C, HD), jnp.float32),
        pltpu.SemaphoreType.DMA,
        pltpu.SemaphoreType.DMA,
    ],
)
def _edge_kernel(xs_hbm, src_hbm, dst_hbm, zeros_hbm, out_hbm,
                 acc_s, srcA, dstA, srcB, dstB, rows_v, semg, semi):
    cid = lax.axis_index("c")
    sid = lax.axis_index("s")
    wid = cid * NS + sid

    row0 = sid * RPT
    pltpu.sync_copy(zeros_hbm.at[pl.ds(row0, RPT)], acc_s.at[pl.ds(row0, RPT)])
    plsc.subcore_barrier()

    ebase = wid * EPTP
    pltpu.sync_copy(src_hbm.at[pl.ds(ebase, C)], srcA)
    pltpu.sync_copy(dst_hbm.at[pl.ds(ebase, C)], dstA)

    def gs(src_v, dst_v):
        pltpu.async_copy(xs_hbm.at[src_v], rows_v, semg).wait()
        pltpu.sync_copy(rows_v, acc_s.at[dst_v], add=True)

    def body(i, carry):
        j0 = 2 * i
        b1 = ebase + (j0 + 1) * C
        c1 = pltpu.async_copy(src_hbm.at[pl.ds(b1, C)], srcB, semi)
        c2 = pltpu.async_copy(dst_hbm.at[pl.ds(b1, C)], dstB, semi)
        gs(srcA, dstA)
        c1.wait()
        c2.wait()
        b2 = ebase + (j0 + 2) * C
        c3 = pltpu.async_copy(src_hbm.at[pl.ds(b2, C)], srcA, semi)
        c4 = pltpu.async_copy(dst_hbm.at[pl.ds(b2, C)], dstA, semi)
        gs(srcB, dstB)
        c3.wait()
        c4.wait()
        return carry

    lax.fori_loop(0, (NCHUNK - 2) // 2, body, 0)
    bl = ebase + (NCHUNK - 1) * C
    c1 = pltpu.async_copy(src_hbm.at[pl.ds(bl, C)], srcB, semi)
    c2 = pltpu.async_copy(dst_hbm.at[pl.ds(bl, C)], dstB, semi)
    gs(srcA, dstA)
    c1.wait()
    c2.wait()
    gs(srcB, dstB)

    plsc.subcore_barrier()
    pltpu.sync_copy(acc_s.at[pl.ds(row0, RPT)], out_hbm.at[cid, pl.ds(row0, RPT)])


def _prep_body(degp_ref, x_ref, W1_ref, dinv_ref, xs_ref):
    ones = jnp.ones((NW, 1), jnp.float32)
    deg = lax.dot_general(degp_ref[...], ones, (((0,), (0,)), ((), ())))
    dinv = lax.rsqrt(deg + 1.0)
    xw = jnp.dot(x_ref[...], W1_ref[...], preferred_element_type=jnp.float32)
    dinv_ref[...] = dinv
    xs_ref[pl.ds(0, N)] = xw * dinv
    xs_ref[pl.ds(N, NPAD - N)] = jnp.zeros((NPAD - N, HD), jnp.float32)


_tc_prep = pl.pallas_call(
    _prep_body,
    out_shape=[
        jax.ShapeDtypeStruct((N, 1), jnp.float32),
        jax.ShapeDtypeStruct((NPAD, HD), jnp.float32),
    ],
)


def _layer_body(acc_ref, xs_ref, dinv_ref, b_ref, W_ref, out_ref):
    dinv = dinv_ref[...]
    acc = (acc_ref[0] + acc_ref[1])[:N]
    h = dinv * (acc + xs_ref[...][:N]) + b_ref[...]
    h = jnp.maximum(h, 0.0)
    out_ref[pl.ds(0, N)] = jnp.dot(h, W_ref[...],
                                   preferred_element_type=jnp.float32) * dinv
    out_ref[pl.ds(N, NPAD - N)] = jnp.zeros((NPAD - N, HD), jnp.float32)


_tc_layer = pl.pallas_call(
    _layer_body,
    out_shape=jax.ShapeDtypeStruct((NPAD, HD), jnp.float32),
)


def _final_body(acc_ref, xs_ref, dinv_ref, b_ref, batch_ref, gW_ref, gb_ref,
                Wr_ref, br_ref, out_ref):
    dinv = dinv_ref[...]
    acc = (acc_ref[0] + acc_ref[1])[:N]
    h = dinv * (acc + xs_ref[...][:N]) + b_ref[...]
    gate = jnp.dot(h, gW_ref[...], preferred_element_type=jnp.float32)
    gate = gate + gb_ref[...]

    gid = lax.broadcasted_iota(jnp.int32, (N, G), 1)
    M = batch_ref[...] == gid
    Mf = M.astype(jnp.float32)
    neg = jnp.float32(-1e30)
    gmax = jnp.max(jnp.where(M, gate, neg), axis=0, keepdims=True)
    gmax_n = jnp.sum(Mf * gmax, axis=1, keepdims=True)
    gexp = jnp.exp(gate - gmax_n)
    gsum = jnp.sum(Mf * gexp, axis=0, keepdims=True)
    gsum_n = jnp.sum(Mf * gsum, axis=1, keepdims=True)
    alpha = gexp / gsum_n
    pooled = lax.dot_general(Mf, alpha * h, (((0,), (0,)), ((), ())))
    out_ref[...] = jnp.tanh(
        jnp.dot(pooled, Wr_ref[...], preferred_element_type=jnp.float32)
        + br_ref[...])


_tc_final = pl.pallas_call(
    _final_body,
    out_shape=jax.ShapeDtypeStruct((G, F), jnp.float32),
)


def kernel(x, edge_index, batch, W1, b1, W2, b2, W3, b3,
           gate_W, gate_b, Wr, br):
    src = edge_index[0]
    dst = edge_index[1]
    zeros = jnp.zeros((NPAD, HD), jnp.float32)

    pad_row = (jnp.arange(EPTP - EPT, dtype=jnp.int32) % (NPAD - N)) + N
    pad = jnp.broadcast_to(pad_row, (NW, EPTP - EPT))
    src_p = jnp.concatenate([src.reshape(NW, EPT), pad], 1).reshape(NW * EPTP)
    dst_p = jnp.concatenate([dst.reshape(NW, EPT), pad], 1).reshape(NW * EPTP)

    degp = _deg_kernel(dst).reshape(NW, N)
    dinv, xs1 = _tc_prep(degp, x, W1)
    acc1 = _edge_kernel(xs1, src_p, dst_p, zeros)
    xs2 = _tc_layer(acc1, xs1, dinv, b1.reshape(1, HD), W2)
    acc2 = _edge_kernel(xs2, src_p, dst_p, zeros)
    xs3 = _tc_layer(acc2, xs2, dinv, b2.reshape(1, HD), W3)
    acc3 = _edge_kernel(xs3, src_p, dst_p, zeros)
    out = _tc_final(acc3, xs3, dinv, b3.reshape(1, HD),
                    batch.reshape(N, 1), gate_W, gate_b.reshape(1, 1),
                    Wr, br.reshape(1, F))
    return out


# speedup vs baseline: 1.0913x; 1.0913x over previous
import functools

import jax
import jax.numpy as jnp
from jax import lax
from jax.experimental import pallas as pl
from jax.experimental.pallas import tpu as pltpu
from jax.experimental.pallas import tpu_sc as plsc

N = 10000
E = 320000
IN = 128
HD = 128
G = 64
F = 3

NC = 2
NS = 16
NW = NC * NS
EPT = E // NW
C = 80
NCHUNK = 128
EPTP = NCHUNK * C
NPAD = 10240
RPT = NPAD // NS

_mesh = plsc.VectorSubcoreMesh(core_axis_name="c", subcore_axis_name="s")


@functools.partial(
    pl.kernel,
    out_type=jax.ShapeDtypeStruct((NW * N,), jnp.float32),
    mesh=_mesh,
    scratch_types=[
        pltpu.VMEM((N,), jnp.float32),
        pltpu.VMEM((EPT,), jnp.int32),
    ],
    compiler_params=pltpu.CompilerParams(needs_layout_passes=False),
)
def _deg_kernel(dst_hbm, out_hbm, deg_v, idx_v):
    cid = lax.axis_index("c")
    sid = lax.axis_index("s")
    wid = cid * NS + sid

    zero16 = jnp.zeros((16,), jnp.float32)

    def zbody(i, carry):
        deg_v[pl.ds(i * 16, 16)] = zero16
        return carry

    lax.fori_loop(0, N // 16, zbody, 0)

    pltpu.sync_copy(dst_hbm.at[pl.ds(wid * EPT, EPT)], idx_v)

    ones16 = jnp.ones((16,), jnp.float32)

    def body(i, carry):
        idx16 = idx_v[pl.ds(i * 16, 16)]
        plsc.addupdate_scatter(deg_v, [idx16], ones16)
        return carry

    lax.fori_loop(0, EPT // 16, body, 0)
    pltpu.sync_copy(deg_v, out_hbm.at[pl.ds(wid * N, N)])


@functools.partial(
    pl.kernel,
    out_type=jax.ShapeDtypeStruct((NC, NPAD, HD), jnp.float32),
    mesh=_mesh,
    scratch_types=[
        pltpu.VMEM_SHARED((NPAD, HD), jnp.float32),
        pltpu.VMEM((C,), jnp.int32),
        pltpu.VMEM((C,), jnp.int32),
        pltpu.VMEM((C,), jnp.int32),
        pltpu.VMEM((C,), jnp.int32),
        pltpu.VMEM((C, HD), jnp.float32),
        pltpu.SemaphoreType.DMA,
        pltpu.SemaphoreType.DMA,
    ],
)
def _edge_kernel(xs_hbm, src_hbm, dst_hbm, zeros_hbm, out_hbm,
                 acc_s, srcA, dstA, srcB, dstB, rows_v, semg, semi):
    cid = lax.axis_index("c")
    sid = lax.axis_index("s")
    wid = cid * NS + sid

    row0 = sid * RPT
    pltpu.sync_copy(zeros_hbm.at[pl.ds(row0, RPT)], acc_s.at[pl.ds(row0, RPT)])
    plsc.subcore_barrier()

    ebase = wid * EPTP
    pltpu.sync_copy(src_hbm.at[pl.ds(ebase, C)], srcA)
    pltpu.sync_copy(dst_hbm.at[pl.ds(ebase, C)], dstA)

    def gs(src_v, dst_v):
        pltpu.async_copy(xs_hbm.at[src_v], rows_v, semg).wait()
        pltpu.sync_copy(rows_v, acc_s.at[dst_v], add=True)

    def body(i, carry):
        j0 = 2 * i
        b1 = ebase + (j0 + 1) * C
        c1 = pltpu.async_copy(src_hbm.at[pl.ds(b1, C)], srcB, semi)
        c2 = pltpu.async_copy(dst_hbm.at[pl.ds(b1, C)], dstB, semi)
        gs(srcA, dstA)
        c1.wait()
        c2.wait()
        b2 = ebase + (j0 + 2) * C
        c3 = pltpu.async_copy(src_hbm.at[pl.ds(b2, C)], srcA, semi)
        c4 = pltpu.async_copy(dst_hbm.at[pl.ds(b2, C)], dstA, semi)
        gs(srcB, dstB)
        c3.wait()
        c4.wait()
        return carry

    lax.fori_loop(0, (NCHUNK - 2) // 2, body, 0)
    bl = ebase + (NCHUNK - 1) * C
    c1 = pltpu.async_copy(src_hbm.at[pl.ds(bl, C)], srcB, semi)
    c2 = pltpu.async_copy(dst_hbm.at[pl.ds(bl, C)], dstB, semi)
    gs(srcA, dstA)
    c1.wait()
    c2.wait()
    gs(srcB, dstB)

    plsc.subcore_barrier()
    pltpu.sync_copy(acc_s.at[pl.ds(row0, RPT)], out_hbm.at[cid, pl.ds(row0, RPT)])


def _prep_body(degp_ref, x_ref, W1_ref, dinv_ref, xs_ref):
    ones = jnp.ones((NW, 1), jnp.float32)
    deg = lax.dot_general(degp_ref[...], ones, (((0,), (0,)), ((), ())))
    dinv = lax.rsqrt(deg + 1.0)
    xw = jnp.dot(x_ref[...], W1_ref[...], preferred_element_type=jnp.float32)
    dinv_ref[...] = dinv
    xs_ref[pl.ds(0, N)] = xw * dinv
    xs_ref[pl.ds(N, NPAD - N)] = jnp.zeros((NPAD - N, HD), jnp.float32)


_tc_prep = pl.pallas_call(
    _prep_body,
    out_shape=[
        jax.ShapeDtypeStruct((N, 1), jnp.float32),
        jax.ShapeDtypeStruct((NPAD, HD), jnp.float32),
    ],
)


def _layer_body(acc_ref, xs_ref, dinv_ref, b_ref, W_ref, out_ref):
    dinv = dinv_ref[...]
    acc = (acc_ref[0] + acc_ref[1])[:N]
    h = dinv * (acc + xs_ref[...][:N]) + b_ref[...]
    h = jnp.maximum(h, 0.0)
    out_ref[pl.ds(0, N)] = jnp.dot(h, W_ref[...],
                                   preferred_element_type=jnp.float32) * dinv
    out_ref[pl.ds(N, NPAD - N)] = jnp.zeros((NPAD - N, HD), jnp.float32)


_tc_layer = pl.pallas_call(
    _layer_body,
    out_shape=jax.ShapeDtypeStruct((NPAD, HD), jnp.float32),
)


def _final_body(acc_ref, xs_ref, dinv_ref, b_ref, batch_ref, gW_ref, gb_ref,
                Wr_ref, br_ref, out_ref):
    dinv = dinv_ref[...]
    acc = (acc_ref[0] + acc_ref[1])[:N]
    h = dinv * (acc + xs_ref[...][:N]) + b_ref[...]
    gate = jnp.dot(h, gW_ref[...], preferred_element_type=jnp.float32)
    gate = gate + gb_ref[...]

    gid = lax.broadcasted_iota(jnp.int32, (N, G), 1)
    M = batch_ref[...] == gid
    Mf = M.astype(jnp.float32)
    neg = jnp.float32(-1e30)
    gmax = jnp.max(jnp.where(M, gate, neg), axis=0, keepdims=True)
    gmax_n = jnp.sum(Mf * gmax, axis=1, keepdims=True)
    gexp = jnp.exp(gate - gmax_n)
    gsum = jnp.sum(Mf * gexp, axis=0, keepdims=True)
    gsum_n = jnp.sum(Mf * gsum, axis=1, keepdims=True)
    alpha = gexp / gsum_n
    pooled = lax.dot_general(Mf, alpha * h, (((0,), (0,)), ((), ())))
    out_ref[...] = jnp.tanh(
        jnp.dot(pooled, Wr_ref[...], preferred_element_type=jnp.float32)
        + br_ref[...])


_tc_final = pl.pallas_call(
    _final_body,
    out_shape=jax.ShapeDtypeStruct((G, F), jnp.float32),
)


def kernel(x, edge_index, batch, W1, b1, W2, b2, W3, b3,
           gate_W, gate_b, Wr, br):
    src = edge_index[0]
    dst = edge_index[1]
    zeros = jnp.zeros((NPAD, HD), jnp.float32)

    pad_row = (jnp.arange(EPTP - EPT, dtype=jnp.int32) % (NPAD - N)) + N
    pad = jnp.broadcast_to(pad_row, (NW, EPTP - EPT))
    src_p = jnp.concatenate([src.reshape(NW, EPT), pad], 1).reshape(NW * EPTP)
    dst_p = jnp.concatenate([dst.reshape(NW, EPT), pad], 1).reshape(NW * EPTP)

    degp = _deg_kernel(dst).reshape(NW, N)
    dinv, xs1 = _tc_prep(degp, x, W1)
    acc1 = _edge_kernel(xs1, src_p, dst_p, zeros)
    xs2 = _tc_layer(acc1, xs1, dinv, b1.reshape(1, HD), W2)
    acc2 = _edge_kernel(xs2, src_p, dst_p, zeros)
    xs3 = _tc_layer(acc2, xs2, dinv, b2.reshape(1, HD), W3)
    acc3 = _edge_kernel(xs3, src_p, dst_p, zeros)
    out = _tc_final(acc3, xs3, dinv, b3.reshape(1, HD),
                    batch.reshape(N, 1), gate_W, gate_b.reshape(1, 1),
                    Wr, br.reshape(1, F))
    return out


# speedup vs baseline: 1.3864x; 1.2705x over previous
import functools

import jax
import jax.numpy as jnp
from jax import lax
from jax.experimental import pallas as pl
from jax.experimental.pallas import tpu as pltpu
from jax.experimental.pallas import tpu_sc as plsc

N = 10000
E = 320000
IN = 128
HD = 128
G = 64
F = 3

NC = 2
NS = 16
NW = NC * NS
EPT = E // NW
C = 80
NCHUNK = 128
EPTP = NCHUNK * C
NPAD = 10240
RPT = NPAD // NS

_mesh = plsc.VectorSubcoreMesh(core_axis_name="c", subcore_axis_name="s")


@functools.partial(
    pl.kernel,
    out_type=jax.ShapeDtypeStruct((NW * N,), jnp.float32),
    mesh=_mesh,
    scratch_types=[
        pltpu.VMEM((N,), jnp.float32),
        pltpu.VMEM((EPT,), jnp.int32),
    ],
    compiler_params=pltpu.CompilerParams(needs_layout_passes=False),
)
def _deg_kernel(dst_hbm, out_hbm, deg_v, idx_v):
    cid = lax.axis_index("c")
    sid = lax.axis_index("s")
    wid = cid * NS + sid

    zero16 = jnp.zeros((16,), jnp.float32)

    def zbody(i, carry):
        deg_v[pl.ds(i * 16, 16)] = zero16
        return carry

    lax.fori_loop(0, N // 16, zbody, 0)

    pltpu.sync_copy(dst_hbm.at[pl.ds(wid * EPT, EPT)], idx_v)

    ones16 = jnp.ones((16,), jnp.float32)

    def body(i, carry):
        idx16 = idx_v[pl.ds(i * 16, 16)]
        plsc.addupdate_scatter(deg_v, [idx16], ones16)
        return carry

    lax.fori_loop(0, EPT // 16, body, 0)
    pltpu.sync_copy(deg_v, out_hbm.at[pl.ds(wid * N, N)])


@functools.partial(
    pl.kernel,
    out_type=jax.ShapeDtypeStruct((NC, NPAD, HD), jnp.float32),
    mesh=_mesh,
    scratch_types=[
        pltpu.VMEM_SHARED((NPAD, HD), jnp.float32),
        pltpu.VMEM((C,), jnp.int32),
        pltpu.VMEM((C,), jnp.int32),
        pltpu.VMEM((C,), jnp.int32),
        pltpu.VMEM((C,), jnp.int32),
        pltpu.VMEM((C, HD), jnp.float32),
        pltpu.VMEM((C, HD), jnp.float32),
        pltpu.SemaphoreType.DMA,
        pltpu.SemaphoreType.DMA,
        pltpu.SemaphoreType.DMA,
    ],
)
def _edge_kernel(xs_hbm, src_hbm, dst_hbm, zeros_hbm, out_hbm,
                 acc_s, srcA, dstA, srcB, dstB, rows0_v, rows1_v,
                 semg0, semg1, semi):
    cid = lax.axis_index("c")
    sid = lax.axis_index("s")
    wid = cid * NS + sid

    row0 = sid * RPT
    pltpu.sync_copy(zeros_hbm.at[pl.ds(row0, RPT)], acc_s.at[pl.ds(row0, RPT)])
    plsc.subcore_barrier()

    ebase = wid * EPTP
    pltpu.sync_copy(src_hbm.at[pl.ds(ebase, C)], srcA)
    pltpu.sync_copy(dst_hbm.at[pl.ds(ebase, C)], dstA)
    g0 = pltpu.async_copy(xs_hbm.at[srcA], rows0_v, semg0)
    pltpu.sync_copy(src_hbm.at[pl.ds(ebase + C, C)], srcB)
    pltpu.sync_copy(dst_hbm.at[pl.ds(ebase + C, C)], dstB)
    g0.wait()

    def body(i, carry):
        j0 = 2 * i
        g1 = pltpu.async_copy(xs_hbm.at[srcB], rows1_v, semg1)
        pltpu.sync_copy(rows0_v, acc_s.at[dstA], add=True)
        b2 = ebase + (j0 + 2) * C
        i1 = pltpu.async_copy(src_hbm.at[pl.ds(b2, C)], srcA, semi)
        i2 = pltpu.async_copy(dst_hbm.at[pl.ds(b2, C)], dstA, semi)
        g1.wait()
        i1.wait()
        i2.wait()
        g2 = pltpu.async_copy(xs_hbm.at[srcA], rows0_v, semg0)
        pltpu.sync_copy(rows1_v, acc_s.at[dstB], add=True)
        b3 = ebase + (j0 + 3) * C
        i3 = pltpu.async_copy(src_hbm.at[pl.ds(b3, C)], srcB, semi)
        i4 = pltpu.async_copy(dst_hbm.at[pl.ds(b3, C)], dstB, semi)
        g2.wait()
        i3.wait()
        i4.wait()
        return carry

    lax.fori_loop(0, (NCHUNK - 2) // 2, body, 0)
    gl = pltpu.async_copy(xs_hbm.at[srcB], rows1_v, semg1)
    pltpu.sync_copy(rows0_v, acc_s.at[dstA], add=True)
    gl.wait()
    pltpu.sync_copy(rows1_v, acc_s.at[dstB], add=True)

    plsc.subcore_barrier()
    pltpu.sync_copy(acc_s.at[pl.ds(row0, RPT)], out_hbm.at[cid, pl.ds(row0, RPT)])


def _prep_body(degp_ref, x_ref, W1_ref, dinv_ref, xs_ref):
    ones = jnp.ones((NW, 1), jnp.float32)
    deg = lax.dot_general(degp_ref[...], ones, (((0,), (0,)), ((), ())))
    dinv = lax.rsqrt(deg + 1.0)
    xw = jnp.dot(x_ref[...], W1_ref[...], preferred_element_type=jnp.float32)
    dinv_ref[...] = dinv
    xs_ref[pl.ds(0, N)] = xw * dinv
    xs_ref[pl.ds(N, NPAD - N)] = jnp.zeros((NPAD - N, HD), jnp.float32)


_tc_prep = pl.pallas_call(
    _prep_body,
    out_shape=[
        jax.ShapeDtypeStruct((N, 1), jnp.float32),
        jax.ShapeDtypeStruct((NPAD, HD), jnp.float32),
    ],
)


def _layer_body(acc_ref, xs_ref, dinv_ref, b_ref, W_ref, out_ref):
    dinv = dinv_ref[...]
    acc = (acc_ref[0] + acc_ref[1])[:N]
    h = dinv * (acc + xs_ref[...][:N]) + b_ref[...]
    h = jnp.maximum(h, 0.0)
    out_ref[pl.ds(0, N)] = jnp.dot(h, W_ref[...],
                                   preferred_element_type=jnp.float32) * dinv
    out_ref[pl.ds(N, NPAD - N)] = jnp.zeros((NPAD - N, HD), jnp.float32)


_tc_layer = pl.pallas_call(
    _layer_body,
    out_shape=jax.ShapeDtypeStruct((NPAD, HD), jnp.float32),
)


def _final_body(acc_ref, xs_ref, dinv_ref, b_ref, batch_ref, gW_ref, gb_ref,
                Wr_ref, br_ref, out_ref):
    dinv = dinv_ref[...]
    acc = (acc_ref[0] + acc_ref[1])[:N]
    h = dinv * (acc + xs_ref[...][:N]) + b_ref[...]
    gate = jnp.dot(h, gW_ref[...], preferred_element_type=jnp.float32)
    gate = gate + gb_ref[...]

    gid = lax.broadcasted_iota(jnp.int32, (N, G), 1)
    M = batch_ref[...] == gid
    Mf = M.astype(jnp.float32)
    neg = jnp.float32(-1e30)
    gmax = jnp.max(jnp.where(M, gate, neg), axis=0, keepdims=True)
    gmax_n = jnp.sum(Mf * gmax, axis=1, keepdims=True)
    gexp = jnp.exp(gate - gmax_n)
    gsum = jnp.sum(Mf * gexp, axis=0, keepdims=True)
    gsum_n = jnp.sum(Mf * gsum, axis=1, keepdims=True)
    alpha = gexp / gsum_n
    pooled = lax.dot_general(Mf, alpha * h, (((0,), (0,)), ((), ())))
    out_ref[...] = jnp.tanh(
        jnp.dot(pooled, Wr_ref[...], preferred_element_type=jnp.float32)
        + br_ref[...])


_tc_final = pl.pallas_call(
    _final_body,
    out_shape=jax.ShapeDtypeStruct((G, F), jnp.float32),
)


def kernel(x, edge_index, batch, W1, b1, W2, b2, W3, b3,
           gate_W, gate_b, Wr, br):
    src = edge_index[0]
    dst = edge_index[1]
    zeros = jnp.zeros((NPAD, HD), jnp.float32)

    pad_row = (jnp.arange(EPTP - EPT, dtype=jnp.int32) % (NPAD - N)) + N
    pad = jnp.broadcast_to(pad_row, (NW, EPTP - EPT))
    src_p = jnp.concatenate([src.reshape(NW, EPT), pad], 1).reshape(NW * EPTP)
    dst_p = jnp.concatenate([dst.reshape(NW, EPT), pad], 1).reshape(NW * EPTP)

    degp = _deg_kernel(dst).reshape(NW, N)
    dinv, xs1 = _tc_prep(degp, x, W1)
    acc1 = _edge_kernel(xs1, src_p, dst_p, zeros)
    xs2 = _tc_layer(acc1, xs1, dinv, b1.reshape(1, HD), W2)
    acc2 = _edge_kernel(xs2, src_p, dst_p, zeros)
    xs3 = _tc_layer(acc2, xs2, dinv, b2.reshape(1, HD), W3)
    acc3 = _edge_kernel(xs3, src_p, dst_p, zeros)
    out = _tc_final(acc3, xs3, dinv, b3.reshape(1, HD),
                    batch.reshape(N, 1), gate_W, gate_b.reshape(1, 1),
                    Wr, br.reshape(1, F))
    return out


# speedup vs baseline: 1.6252x; 1.1722x over previous
import functools

import jax
import jax.numpy as jnp
from jax import lax
from jax.experimental import pallas as pl
from jax.experimental.pallas import tpu as pltpu
from jax.experimental.pallas import tpu_sc as plsc

N = 10000
E = 320000
IN = 128
HD = 128
G = 64
F = 3

NC = 2
NS = 16
NW = NC * NS
EPT = E // NW
C = 128
NCHUNK = 80
EPTP = NCHUNK * C
NPAD = 10240
RPT = NPAD // NS

_mesh = plsc.VectorSubcoreMesh(core_axis_name="c", subcore_axis_name="s")


@functools.partial(
    pl.kernel,
    out_type=jax.ShapeDtypeStruct((NW * N,), jnp.float32),
    mesh=_mesh,
    scratch_types=[
        pltpu.VMEM((N,), jnp.float32),
        pltpu.VMEM((EPT,), jnp.int32),
    ],
    compiler_params=pltpu.CompilerParams(needs_layout_passes=False),
)
def _deg_kernel(dst_hbm, out_hbm, deg_v, idx_v):
    cid = lax.axis_index("c")
    sid = lax.axis_index("s")
    wid = cid * NS + sid

    zero16 = jnp.zeros((16,), jnp.float32)

    def zbody(i, carry):
        deg_v[pl.ds(i * 16, 16)] = zero16
        return carry

    lax.fori_loop(0, N // 16, zbody, 0)

    pltpu.sync_copy(dst_hbm.at[pl.ds(wid * EPT, EPT)], idx_v)

    ones16 = jnp.ones((16,), jnp.float32)

    def body(i, carry):
        idx16 = idx_v[pl.ds(i * 16, 16)]
        plsc.addupdate_scatter(deg_v, [idx16], ones16)
        return carry

    lax.fori_loop(0, EPT // 16, body, 0)
    pltpu.sync_copy(deg_v, out_hbm.at[pl.ds(wid * N, N)])


@functools.partial(
    pl.kernel,
    out_type=jax.ShapeDtypeStruct((NC, NPAD, HD), jnp.float32),
    mesh=_mesh,
    scratch_types=[
        pltpu.VMEM_SHARED((NPAD, HD), jnp.float32),
        pltpu.VMEM((C,), jnp.int32),
        pltpu.VMEM((C,), jnp.int32),
        pltpu.VMEM((C,), jnp.int32),
        pltpu.VMEM((C,), jnp.int32),
        pltpu.VMEM((C, HD), jnp.float32),
        pltpu.VMEM((C, HD), jnp.float32),
        pltpu.SemaphoreType.DMA,
        pltpu.SemaphoreType.DMA,
        pltpu.SemaphoreType.DMA,
    ],
)
def _edge_kernel(xs_hbm, src_hbm, dst_hbm, zeros_hbm, out_hbm,
                 acc_s, srcA, dstA, srcB, dstB, rows0_v, rows1_v,
                 semg0, semg1, semi):
    cid = lax.axis_index("c")
    sid = lax.axis_index("s")
    wid = cid * NS + sid

    row0 = sid * RPT
    pltpu.sync_copy(zeros_hbm.at[pl.ds(row0, RPT)], acc_s.at[pl.ds(row0, RPT)])
    plsc.subcore_barrier()

    ebase = wid * EPTP
    pltpu.sync_copy(src_hbm.at[pl.ds(ebase, C)], srcA)
    pltpu.sync_copy(dst_hbm.at[pl.ds(ebase, C)], dstA)
    g0 = pltpu.async_copy(xs_hbm.at[srcA], rows0_v, semg0)
    pltpu.sync_copy(src_hbm.at[pl.ds(ebase + C, C)], srcB)
    pltpu.sync_copy(dst_hbm.at[pl.ds(ebase + C, C)], dstB)
    g0.wait()

    def body(i, carry):
        j0 = 2 * i
        g1 = pltpu.async_copy(xs_hbm.at[srcB], rows1_v, semg1)
        pltpu.sync_copy(rows0_v, acc_s.at[dstA], add=True)
        b2 = ebase + (j0 + 2) * C
        i1 = pltpu.async_copy(src_hbm.at[pl.ds(b2, C)], srcA, semi)
        i2 = pltpu.async_copy(dst_hbm.at[pl.ds(b2, C)], dstA, semi)
        g1.wait()
        i1.wait()
        i2.wait()
        g2 = pltpu.async_copy(xs_hbm.at[srcA], rows0_v, semg0)
        pltpu.sync_copy(rows1_v, acc_s.at[dstB], add=True)
        b3 = ebase + (j0 + 3) * C
        i3 = pltpu.async_copy(src_hbm.at[pl.ds(b3, C)], srcB, semi)
        i4 = pltpu.async_copy(dst_hbm.at[pl.ds(b3, C)], dstB, semi)
        g2.wait()
        i3.wait()
        i4.wait()
        return carry

    lax.fori_loop(0, (NCHUNK - 2) // 2, body, 0)
    gl = pltpu.async_copy(xs_hbm.at[srcB], rows1_v, semg1)
    pltpu.sync_copy(rows0_v, acc_s.at[dstA], add=True)
    gl.wait()
    pltpu.sync_copy(rows1_v, acc_s.at[dstB], add=True)

    plsc.subcore_barrier()
    pltpu.sync_copy(acc_s.at[pl.ds(row0, RPT)], out_hbm.at[cid, pl.ds(row0, RPT)])


def _prep_body(degp_ref, x_ref, W1_ref, dinv_ref, xs_ref):
    ones = jnp.ones((NW, 1), jnp.float32)
    deg = lax.dot_general(degp_ref[...], ones, (((0,), (0,)), ((), ())))
    dinv = lax.rsqrt(deg + 1.0)
    xw = jnp.dot(x_ref[...], W1_ref[...], preferred_element_type=jnp.float32)
    dinv_ref[...] = dinv
    xs_ref[pl.ds(0, N)] = xw * dinv
    xs_ref[pl.ds(N, NPAD - N)] = jnp.zeros((NPAD - N, HD), jnp.float32)


_tc_prep = pl.pallas_call(
    _prep_body,
    out_shape=[
        jax.ShapeDtypeStruct((N, 1), jnp.float32),
        jax.ShapeDtypeStruct((NPAD, HD), jnp.float32),
    ],
)


def _layer_body(acc_ref, xs_ref, dinv_ref, b_ref, W_ref, out_ref):
    dinv = dinv_ref[...]
    acc = (acc_ref[0] + acc_ref[1])[:N]
    h = dinv * (acc + xs_ref[...][:N]) + b_ref[...]
    h = jnp.maximum(h, 0.0)
    out_ref[pl.ds(0, N)] = jnp.dot(h, W_ref[...],
                                   preferred_element_type=jnp.float32) * dinv
    out_ref[pl.ds(N, NPAD - N)] = jnp.zeros((NPAD - N, HD), jnp.float32)


_tc_layer = pl.pallas_call(
    _layer_body,
    out_shape=jax.ShapeDtypeStruct((NPAD, HD), jnp.float32),
)


def _final_body(acc_ref, xs_ref, dinv_ref, b_ref, batch_ref, gW_ref, gb_ref,
                Wr_ref, br_ref, out_ref):
    dinv = dinv_ref[...]
    acc = (acc_ref[0] + acc_ref[1])[:N]
    h = dinv * (acc + xs_ref[...][:N]) + b_ref[...]
    gate = jnp.dot(h, gW_ref[...], preferred_element_type=jnp.float32)
    gate = gate + gb_ref[...]

    gid = lax.broadcasted_iota(jnp.int32, (N, G), 1)
    M = batch_ref[...] == gid
    Mf = M.astype(jnp.float32)
    neg = jnp.float32(-1e30)
    gmax = jnp.max(jnp.where(M, gate, neg), axis=0, keepdims=True)
    gmax_n = jnp.sum(Mf * gmax, axis=1, keepdims=True)
    gexp = jnp.exp(gate - gmax_n)
    gsum = jnp.sum(Mf * gexp, axis=0, keepdims=True)
    gsum_n = jnp.sum(Mf * gsum, axis=1, keepdims=True)
    alpha = gexp / gsum_n
    pooled = lax.dot_general(Mf, alpha * h, (((0,), (0,)), ((), ())))
    out_ref[...] = jnp.tanh(
        jnp.dot(pooled, Wr_ref[...], preferred_element_type=jnp.float32)
        + br_ref[...])


_tc_final = pl.pallas_call(
    _final_body,
    out_shape=jax.ShapeDtypeStruct((G, F), jnp.float32),
)


def kernel(x, edge_index, batch, W1, b1, W2, b2, W3, b3,
           gate_W, gate_b, Wr, br):
    src = edge_index[0]
    dst = edge_index[1]
    zeros = jnp.zeros((NPAD, HD), jnp.float32)

    pad_row = (jnp.arange(EPTP - EPT, dtype=jnp.int32) % (NPAD - N)) + N
    pad = jnp.broadcast_to(pad_row, (NW, EPTP - EPT))
    src_p = jnp.concatenate([src.reshape(NW, EPT), pad], 1).reshape(NW * EPTP)
    dst_p = jnp.concatenate([dst.reshape(NW, EPT), pad], 1).reshape(NW * EPTP)

    degp = _deg_kernel(dst).reshape(NW, N)
    dinv, xs1 = _tc_prep(degp, x, W1)
    acc1 = _edge_kernel(xs1, src_p, dst_p, zeros)
    xs2 = _tc_layer(acc1, xs1, dinv, b1.reshape(1, HD), W2)
    acc2 = _edge_kernel(xs2, src_p, dst_p, zeros)
    xs3 = _tc_layer(acc2, xs2, dinv, b2.reshape(1, HD), W3)
    acc3 = _edge_kernel(xs3, src_p, dst_p, zeros)
    out = _tc_final(acc3, xs3, dinv, b3.reshape(1, HD),
                    batch.reshape(N, 1), gate_W, gate_b.reshape(1, 1),
                    Wr, br.reshape(1, F))
    return out
